# Initial kernel scaffold; baseline (speedup 1.0000x reference)
import jax
import jax.numpy as jnp
from jax.experimental import pallas as pl


def kernel(pc_source, pc_target, pred_flow, weights):
    raise NotImplementedError("write your pallas kernel here")


# speedup vs baseline: 72.6793x; 72.6793x over previous
import functools

import jax
import jax.numpy as jnp
from jax import lax
from jax.experimental import pallas as pl
from jax.experimental.pallas import tpu as pltpu
from jax.experimental.pallas import tpu_sc as plsc

_B, _N, _K = 4, 4096, 16
_NC, _NS, _L = 2, 16, 16
_NW = _NC * _NS
_QPW = _B * _N // _NW
_WPB = _N // _QPW
_INTER = 8

_mesh = plsc.VectorSubcoreMesh(
    core_axis_name="c", subcore_axis_name="s",
    num_cores=_NC, num_subcores=_NS,
)


@functools.partial(
    pl.kernel,
    out_type=jax.ShapeDtypeStruct((_B * _N, _K), jnp.float32),
    mesh=_mesh,
    compiler_params=pltpu.CompilerParams(needs_layout_passes=False),
    scratch_types=[
        pltpu.VMEM((_QPW,), jnp.float32),
        pltpu.VMEM((_QPW,), jnp.float32),
        pltpu.VMEM((_QPW,), jnp.float32),
        pltpu.VMEM((_N,), jnp.float32),
        pltpu.VMEM((_N,), jnp.float32),
        pltpu.VMEM((_N,), jnp.float32),
        pltpu.VMEM((_N,), jnp.float32),
        pltpu.VMEM((_QPW, _K), jnp.float32),
    ],
)
def _sc_topk(qx, qy, qz, tx, ty, tz, out,
             qx_v, qy_v, qz_v, tx_v, ty_v, tz_v, tn_v, out_v):
    wid = lax.axis_index("s") * _NC + lax.axis_index("c")
    b = wid // _WPB
    qbase = wid * _QPW
    tbase = b * _N

    pltpu.sync_copy(tx.at[pl.ds(tbase, _N)], tx_v)
    pltpu.sync_copy(ty.at[pl.ds(tbase, _N)], ty_v)
    pltpu.sync_copy(tz.at[pl.ds(tbase, _N)], tz_v)
    pltpu.sync_copy(qx.at[pl.ds(qbase, _QPW)], qx_v)
    pltpu.sync_copy(qy.at[pl.ds(qbase, _QPW)], qy_v)
    pltpu.sync_copy(qz.at[pl.ds(qbase, _QPW)], qz_v)

    def tn_body(j, _):
        off = pl.multiple_of(j * _L, _L)
        a = tx_v[pl.ds(off, _L)]
        c = ty_v[pl.ds(off, _L)]
        d = tz_v[pl.ds(off, _L)]
        tn_v[pl.ds(off, _L)] = a * a + c * c + d * d
        return 0
    lax.fori_loop(0, _N // _L, tn_body, 0)

    def group_body(g, _):
        base = pl.multiple_of(g * _L, _L)
        qxv = qx_v[pl.ds(base, _L)]
        qyv = qy_v[pl.ds(base, _L)]
        qzv = qz_v[pl.ds(base, _L)]
        for h in range(_L // _INTER):
            q2x, q2y, q2z, qn = [], [], [], []
            for u in range(_INTER):
                sx = qxv[h * _INTER + u]
                sy = qyv[h * _INTER + u]
                sz = qzv[h * _INTER + u]
                q2x.append(jnp.full((_L,), sx * 2.0, jnp.float32))
                q2y.append(jnp.full((_L,), sy * 2.0, jnp.float32))
                q2z.append(jnp.full((_L,), sz * 2.0, jnp.float32))
                qn.append(sx * sx + sy * sy + sz * sz)

            def j_body(j, bests):
                off = pl.multiple_of(j * _L, _L)
                a = tx_v[pl.ds(off, _L)]
                c = ty_v[pl.ds(off, _L)]
                d = tz_v[pl.ds(off, _L)]
                n = tn_v[pl.ds(off, _L)]
                new = []
                for u in range(_INTER):
                    s = n - (a * q2x[u] + c * q2y[u] + d * q2z[u])
                    cd, _unused = plsc.sort_key_val(s, s, descending=True)
                    m = jnp.minimum(bests[u], cd)
                    ba, _unused = plsc.sort_key_val(m, m)
                    new.append(ba)
                return tuple(new)

            init = tuple(jnp.full((_L,), jnp.inf, jnp.float32)
                         for _u in range(_INTER))
            bests = lax.fori_loop(0, _N // _L, j_body, init)
            for u in range(_INTER):
                i = g * _L + h * _INTER + u
                out_v[i, :] = bests[u] + jnp.full((_L,), qn[u], jnp.float32)
        return 0

    lax.fori_loop(0, _QPW // _L, group_body, 0)
    pltpu.sync_copy(out_v, out.at[pl.ds(qbase, _QPW)])


_RPB = _N * _K // 128
_WRB = _N // 128


def _tc_loss_body(d2_ref, wrep_ref, w_ref, out_ref):
    acc = jnp.float32(0.0)
    for b in range(_B):
        dist = jnp.sqrt(jnp.maximum(d2_ref[b], 0.0))
        num = jnp.sum(dist * wrep_ref[b])
        den = jnp.sum(w_ref[b])
        acc += num / (den * _K)
    out_ref[0, 0] = acc / _B


_tc_loss = pl.pallas_call(
    _tc_loss_body,
    out_shape=jax.ShapeDtypeStruct((1, 1), jnp.float32),
    out_specs=pl.BlockSpec(memory_space=pltpu.SMEM),
)


def kernel(pc_source, pc_target, pred_flow, weights):
    q = pc_source + pred_flow
    qx = q[..., 0].reshape(-1)
    qy = q[..., 1].reshape(-1)
    qz = q[..., 2].reshape(-1)
    tx = pc_target[..., 0].reshape(-1)
    ty = pc_target[..., 1].reshape(-1)
    tz = pc_target[..., 2].reshape(-1)
    d2_top = _sc_topk(qx, qy, qz, tx, ty, tz)
    d2r = d2_top.reshape(_B, _RPB, 128)
    wrep = jnp.repeat(weights, _K, axis=1).reshape(_B, _RPB, 128)
    wr = weights.reshape(_B, _WRB, 128)
    loss = _tc_loss(d2r, wrep, wr)
    return loss.reshape(())
